# Initial kernel scaffold; baseline (speedup 1.0000x reference)
import jax
import jax.numpy as jnp
from jax.experimental import pallas as pl


def kernel(emb, labels, sbj):
    raise NotImplementedError("write your pallas kernel here")


# speedup vs baseline: 209.1152x; 209.1152x over previous
import functools

import jax
import jax.numpy as jnp
from jax import lax
from jax.experimental import pallas as pl
from jax.experimental.pallas import tpu as pltpu
from jax.experimental.pallas import tpu_sc as plsc

B = 16384
D = 256
N_CLASSES = 6
N_SUBJECTS = 100
NPAIR = 640
MARGIN = 1.0
EPS = 1e-6
ROWS = 1024
NBLK = B // ROWS
I_SENT = 2**30
F_BIG = 1e30


def _mine_body(p_ref, idx_ref, cnt_ref, scnt_ref):
    qi = lax.broadcasted_iota(jnp.int32, (ROWS, NPAIR), 1)
    qs = qi // N_CLASSES
    f = jnp.full((NPAIR,), I_SENT, jnp.int32)
    cnt = jnp.zeros((NPAIR,), jnp.int32)
    scnt = jnp.zeros((NPAIR,), jnp.int32)
    for r in range(NBLK):
        pr = p_ref[r, :][:, None]
        jg = r * ROWS + lax.broadcasted_iota(jnp.int32, (ROWS, NPAIR), 0)
        eq = pr == qi
        midx = jnp.where(eq, jg, I_SENT)
        f = jnp.minimum(f, jnp.min(midx, axis=0))
        cnt = cnt + jnp.sum(eq.astype(jnp.int32), axis=0)
        scnt = scnt + jnp.sum((pr // N_CLASSES == qs).astype(jnp.int32), axis=0)
    s = jnp.full((NPAIR,), I_SENT, jnp.int32)
    for r in range(NBLK):
        pr = p_ref[r, :][:, None]
        jg = r * ROWS + lax.broadcasted_iota(jnp.int32, (ROWS, NPAIR), 0)
        midx = jnp.where(pr == qi, jg, I_SENT)
        midx = jnp.where(midx == f[None, :], I_SENT, midx)
        s = jnp.minimum(s, jnp.min(midx, axis=0))
    idx_ref[0:NPAIR] = jnp.minimum(f, B - 1)
    idx_ref[NPAIR:2 * NPAIR] = jnp.minimum(s, B - 1)
    cnt_ref[...] = cnt
    scnt_ref[...] = scnt


def _mine(p2d):
    return pl.pallas_call(
        _mine_body,
        out_shape=(
            jax.ShapeDtypeStruct((2 * NPAIR,), jnp.int32),
            jax.ShapeDtypeStruct((NPAIR,), jnp.int32),
            jax.ShapeDtypeStruct((NPAIR,), jnp.int32),
        ),
    )(p2d)


_ROWS_PER_W = (2 * NPAIR) // 32


@functools.cache
def _make_sc_gather():
    mesh = plsc.VectorSubcoreMesh(core_axis_name="c", subcore_axis_name="s")

    @functools.partial(
        pl.kernel,
        mesh=mesh,
        out_type=jax.ShapeDtypeStruct((2 * NPAIR, D), jnp.float32),
        scratch_types=[
            pltpu.VMEM((_ROWS_PER_W,), jnp.int32),
            pltpu.VMEM((_ROWS_PER_W, D), jnp.float32),
            pltpu.SemaphoreType.DMA,
        ],
    )
    def _g(emb_hbm, idx_hbm, out_hbm, idx_v, rows_v, sem):
        wid = lax.axis_index("s") * 2 + lax.axis_index("c")
        base = wid * _ROWS_PER_W
        pltpu.sync_copy(idx_hbm.at[pl.ds(base, _ROWS_PER_W)], idx_v)
        pltpu.async_copy(emb_hbm.at[idx_v], rows_v, sem).wait()
        pltpu.sync_copy(rows_v, out_hbm.at[pl.ds(base, _ROWS_PER_W)])

    return _g


def _sc_gather(emb, idx):
    return _make_sc_gather()(emb, idx)


def _dist_body(emb_ref, p_ref, a_ref, pos_ref, cnt_ref, scnt_ref, loss_ref,
               minacc):
    step = pl.program_id(0)

    @pl.when(step == 0)
    def _():
        minacc[...] = jnp.full((NPAIR,), F_BIG, jnp.float32)

    E = emb_ref[...]
    G = lax.dot_general(E, a_ref[...], (((1,), (1,)), ((), ())),
                        preferred_element_type=jnp.float32)
    en = jnp.sum(E * E, axis=1)
    val = en[:, None] - 2.0 * G
    pr = p_ref[0, 0, :][:, None]
    qi = lax.broadcasted_iota(jnp.int32, (ROWS, NPAIR), 1)
    mask = (pr // N_CLASSES == qi // N_CLASSES) & (pr != qi)
    masked = jnp.where(mask, val, F_BIG)
    minacc[...] = jnp.minimum(minacc[...], jnp.min(masked, axis=0))

    @pl.when(step == NBLK - 1)
    def _():
        A = a_ref[...]
        P = pos_ref[...]
        an2 = jnp.sum(A * A, axis=1)
        d_an = jnp.sqrt(jnp.maximum(minacc[...] + an2, 0.0))
        dif = A - P + EPS
        d_ap = jnp.sqrt(jnp.sum(dif * dif, axis=1))
        npos = cnt_ref[...]
        nneg = scnt_ref[...] - npos
        valid = (npos >= 2) & (nneg >= 1)
        term = jnp.where(valid, jnp.maximum(d_ap - d_an + MARGIN, 0.0), 0.0)
        total = jnp.sum(term)
        count = jnp.sum(valid.astype(jnp.float32))
        loss = jnp.where(count > 0.0, total / jnp.maximum(count, 1.0),
                         jnp.float32(0.0))
        loss_ref[...] = jnp.broadcast_to(loss, (1, 1))


def _dist(emb, p3, a_rows, p_rows, cnt, scnt):
    return pl.pallas_call(
        _dist_body,
        grid=(NBLK,),
        in_specs=[
            pl.BlockSpec((ROWS, D), lambda s: (s, 0)),
            pl.BlockSpec((1, 1, ROWS), lambda s: (s, 0, 0)),
            pl.BlockSpec((NPAIR, D), lambda s: (0, 0)),
            pl.BlockSpec((NPAIR, D), lambda s: (0, 0)),
            pl.BlockSpec((NPAIR,), lambda s: (0,)),
            pl.BlockSpec((NPAIR,), lambda s: (0,)),
        ],
        out_specs=pl.BlockSpec((1, 1), lambda s: (0, 0)),
        out_shape=jax.ShapeDtypeStruct((1, 1), jnp.float32),
        scratch_shapes=[pltpu.VMEM((NPAIR,), jnp.float32)],
    )(emb, p3, a_rows, p_rows, cnt, scnt)


def kernel(emb, labels, sbj):
    p = sbj * N_CLASSES + labels
    idx, cnt, scnt = _mine(p.reshape(NBLK, ROWS))
    rows = _sc_gather(emb, idx)
    a_rows = rows[:NPAIR]
    p_rows = rows[NPAIR:]
    loss = _dist(emb, p.reshape(NBLK, 1, ROWS), a_rows, p_rows, cnt, scnt)
    return loss.reshape(())


# speedup vs baseline: 255.4175x; 1.2214x over previous
import functools

import jax
import jax.numpy as jnp
from jax import lax
from jax.experimental import pallas as pl
from jax.experimental.pallas import tpu as pltpu
from jax.experimental.pallas import tpu_sc as plsc

B = 16384
D = 256
N_CLASSES = 6
N_SUBJECTS = 100
NPAIR = 640
MARGIN = 1.0
EPS = 1e-6
ROWS = 1024
NBLK = B // ROWS
DROWS = 2048
DNBLK = B // DROWS
I_SENT = 2**30
F_BIG = 1e30


def _mine_body(p_ref, idx_ref, valid2_ref):
    qi = lax.broadcasted_iota(jnp.int32, (ROWS, NPAIR), 1)
    jg = lax.broadcasted_iota(jnp.int32, (ROWS, NPAIR), 0)
    f = jnp.full((NPAIR,), I_SENT, jnp.int32)
    s = jnp.full((NPAIR,), I_SENT, jnp.int32)
    for r in range(NBLK):
        pr = p_ref[r, :][:, None]
        m = jnp.where(pr == qi, jg, I_SENT)
        c1 = jnp.min(m, axis=0)
        c2 = jnp.min(jnp.where(m == c1[None, :], I_SENT, m), axis=0)
        c1 = jnp.where(c1 < I_SENT, c1 + r * ROWS, I_SENT)
        c2 = jnp.where(c2 < I_SENT, c2 + r * ROWS, I_SENT)
        s = jnp.minimum(jnp.maximum(f, c1), jnp.minimum(s, c2))
        f = jnp.minimum(f, c1)
    idx_ref[0:NPAIR] = jnp.minimum(f, B - 1)
    idx_ref[NPAIR:2 * NPAIR] = jnp.minimum(s, B - 1)
    valid2_ref[...] = (s < I_SENT).astype(jnp.int32)


def _mine(p2d):
    return pl.pallas_call(
        _mine_body,
        out_shape=(
            jax.ShapeDtypeStruct((2 * NPAIR,), jnp.int32),
            jax.ShapeDtypeStruct((NPAIR,), jnp.int32),
        ),
    )(p2d)


_ROWS_PER_W = (2 * NPAIR) // 32


@functools.cache
def _make_sc_gather():
    mesh = plsc.VectorSubcoreMesh(core_axis_name="c", subcore_axis_name="s")

    @functools.partial(
        pl.kernel,
        mesh=mesh,
        out_type=jax.ShapeDtypeStruct((2 * NPAIR, D), jnp.float32),
        scratch_types=[
            pltpu.VMEM((_ROWS_PER_W,), jnp.int32),
            pltpu.VMEM((_ROWS_PER_W, D), jnp.float32),
            pltpu.SemaphoreType.DMA,
        ],
    )
    def _g(emb_hbm, idx_hbm, out_hbm, idx_v, rows_v, sem):
        wid = lax.axis_index("s") * 2 + lax.axis_index("c")
        base = wid * _ROWS_PER_W
        pltpu.sync_copy(idx_hbm.at[pl.ds(base, _ROWS_PER_W)], idx_v)
        pltpu.async_copy(emb_hbm.at[idx_v], rows_v, sem).wait()
        pltpu.sync_copy(rows_v, out_hbm.at[pl.ds(base, _ROWS_PER_W)])

    return _g


def _sc_gather(emb, idx):
    return _make_sc_gather()(emb, idx)


def _dist_body(emb_ref, p_ref, a_ref, pos_ref, valid2_ref, loss_ref, minacc):
    step = pl.program_id(0)

    @pl.when(step == 0)
    def _():
        minacc[...] = jnp.full((NPAIR,), F_BIG, jnp.float32)

    E = emb_ref[...]
    G = lax.dot_general(E, a_ref[...], (((1,), (1,)), ((), ())),
                        preferred_element_type=jnp.float32)
    en = jnp.sum(E * E, axis=1)
    val = en[:, None] - 2.0 * G
    pr = p_ref[0, 0, :][:, None]
    qi = lax.broadcasted_iota(jnp.int32, (DROWS, NPAIR), 1)
    qs = lax.broadcasted_iota(jnp.int32, (1, NPAIR), 1) // N_CLASSES
    mask = ((pr // N_CLASSES) == qs) & (pr != qi)
    masked = jnp.where(mask, val, F_BIG)
    minacc[...] = jnp.minimum(minacc[...], jnp.min(masked, axis=0))

    @pl.when(step == DNBLK - 1)
    def _():
        A = a_ref[...]
        P = pos_ref[...]
        an2 = jnp.sum(A * A, axis=1)
        mn = minacc[...]
        d_an = jnp.sqrt(jnp.maximum(mn + an2, 0.0))
        dif = A - P + EPS
        d_ap = jnp.sqrt(jnp.sum(dif * dif, axis=1))
        valid = (valid2_ref[...] > 0) & (mn < 1e29)
        term = jnp.where(valid, jnp.maximum(d_ap - d_an + MARGIN, 0.0), 0.0)
        total = jnp.sum(term)
        count = jnp.sum(valid.astype(jnp.float32))
        loss = jnp.where(count > 0.0, total / jnp.maximum(count, 1.0),
                         jnp.float32(0.0))
        loss_ref[...] = jnp.broadcast_to(loss, (1, 1))


def _dist(emb, p3, a_rows, p_rows, valid2):
    return pl.pallas_call(
        _dist_body,
        grid=(DNBLK,),
        in_specs=[
            pl.BlockSpec((DROWS, D), lambda s: (s, 0)),
            pl.BlockSpec((1, 1, DROWS), lambda s: (s, 0, 0)),
            pl.BlockSpec((NPAIR, D), lambda s: (0, 0)),
            pl.BlockSpec((NPAIR, D), lambda s: (0, 0)),
            pl.BlockSpec((NPAIR,), lambda s: (0,)),
        ],
        out_specs=pl.BlockSpec((1, 1), lambda s: (0, 0)),
        out_shape=jax.ShapeDtypeStruct((1, 1), jnp.float32),
        scratch_shapes=[pltpu.VMEM((NPAIR,), jnp.float32)],
    )(emb, p3, a_rows, p_rows, valid2)


def kernel(emb, labels, sbj):
    p = sbj * N_CLASSES + labels
    idx, valid2 = _mine(p.reshape(NBLK, ROWS))
    rows = _sc_gather(emb, idx)
    a_rows = rows[:NPAIR]
    p_rows = rows[NPAIR:]
    loss = _dist(emb, p.reshape(DNBLK, 1, DROWS), a_rows, p_rows, valid2)
    return loss.reshape(())


# speedup vs baseline: 266.6238x; 1.0439x over previous
import functools

import jax
import jax.numpy as jnp
from jax import lax
from jax.experimental import pallas as pl
from jax.experimental.pallas import tpu as pltpu
from jax.experimental.pallas import tpu_sc as plsc

B = 16384
D = 256
N_CLASSES = 6
N_SUBJECTS = 100
NPAIR = 640
MARGIN = 1.0
EPS = 1e-6
ROWS = 1024
NBLK = B // ROWS
DROWS = 2048
DNBLK = B // DROWS
I_SENT = 2**30
F_BIG = 1e30


def _mine_body(p_ref, idx_ref, valid2_ref):
    qi = lax.broadcasted_iota(jnp.int32, (ROWS, NPAIR), 1)
    jg = lax.broadcasted_iota(jnp.int32, (ROWS, NPAIR), 0)
    f = jnp.full((NPAIR,), I_SENT, jnp.int32)
    s = jnp.full((NPAIR,), I_SENT, jnp.int32)
    for r in range(NBLK):
        pr = p_ref[r, :][:, None]
        m = jnp.where(pr == qi, jg, I_SENT)
        c1 = jnp.min(m, axis=0)
        c2 = jnp.min(jnp.where(m == c1[None, :], I_SENT, m), axis=0)
        c1 = jnp.where(c1 < I_SENT, c1 + r * ROWS, I_SENT)
        c2 = jnp.where(c2 < I_SENT, c2 + r * ROWS, I_SENT)
        s = jnp.minimum(jnp.maximum(f, c1), jnp.minimum(s, c2))
        f = jnp.minimum(f, c1)
    idx_ref[0:NPAIR] = jnp.minimum(f, B - 1)
    idx_ref[NPAIR:2 * NPAIR] = jnp.minimum(s, B - 1)
    valid2_ref[...] = (s < I_SENT).astype(jnp.int32)


def _mine(p2d):
    return pl.pallas_call(
        _mine_body,
        out_shape=(
            jax.ShapeDtypeStruct((2 * NPAIR,), jnp.int32),
            jax.ShapeDtypeStruct((NPAIR,), jnp.int32),
        ),
    )(p2d)


_ROWS_PER_W = (2 * NPAIR) // 32


@functools.cache
def _make_sc_gather():
    mesh = plsc.VectorSubcoreMesh(core_axis_name="c", subcore_axis_name="s")

    @functools.partial(
        pl.kernel,
        mesh=mesh,
        out_type=jax.ShapeDtypeStruct((2 * NPAIR, D), jnp.float32),
        scratch_types=[
            pltpu.VMEM((_ROWS_PER_W,), jnp.int32),
            pltpu.VMEM((_ROWS_PER_W, D), jnp.float32),
            pltpu.SemaphoreType.DMA,
        ],
    )
    def _g(emb_hbm, idx_hbm, out_hbm, idx_v, rows_v, sem):
        wid = lax.axis_index("s") * 2 + lax.axis_index("c")
        base = wid * _ROWS_PER_W
        pltpu.sync_copy(idx_hbm.at[pl.ds(base, _ROWS_PER_W)], idx_v)
        pltpu.async_copy(emb_hbm.at[idx_v], rows_v, sem).wait()
        pltpu.sync_copy(rows_v, out_hbm.at[pl.ds(base, _ROWS_PER_W)])

    return _g


def _sc_gather(emb, idx):
    return _make_sc_gather()(emb, idx)


def _dist_body(emb_ref, p_ref, a_ref, pos_ref, valid2_ref, loss_ref, minacc):
    step = pl.program_id(0)

    @pl.when(step == 0)
    def _():
        minacc[...] = jnp.full((NPAIR,), F_BIG, jnp.float32)

    E = emb_ref[...]
    G = lax.dot_general(E, a_ref[...], (((1,), (1,)), ((), ())),
                        preferred_element_type=jnp.float32)
    en = jnp.sum(E * E, axis=1)
    val = en[:, None] - 2.0 * G
    pr = p_ref[0, 0, :][:, None]
    qi = lax.broadcasted_iota(jnp.int32, (DROWS, NPAIR), 1)
    qs = lax.broadcasted_iota(jnp.int32, (1, NPAIR), 1) // N_CLASSES
    mask = ((pr // N_CLASSES) == qs) & (pr != qi)
    masked = jnp.where(mask, val, F_BIG)
    minacc[...] = jnp.minimum(minacc[...], jnp.min(masked, axis=0))

    @pl.when(step == DNBLK - 1)
    def _():
        A = a_ref[...]
        P = pos_ref[...]
        an2 = jnp.sum(A * A, axis=1)
        mn = minacc[...]
        d_an = jnp.sqrt(jnp.maximum(mn + an2, 0.0))
        dif = A - P + EPS
        d_ap = jnp.sqrt(jnp.sum(dif * dif, axis=1))
        valid = (valid2_ref[...] > 0) & (mn < 1e29)
        term = jnp.where(valid, jnp.maximum(d_ap - d_an + MARGIN, 0.0), 0.0)
        total = jnp.sum(term)
        count = jnp.sum(valid.astype(jnp.float32))
        loss = jnp.where(count > 0.0, total / jnp.maximum(count, 1.0),
                         jnp.float32(0.0))
        loss_ref[...] = jnp.broadcast_to(loss, (1, 1))


def _dist(emb, p3, a_rows, p_rows, valid2):
    return pl.pallas_call(
        _dist_body,
        grid=(DNBLK,),
        in_specs=[
            pl.BlockSpec((DROWS, D), lambda s: (s, 0)),
            pl.BlockSpec((1, 1, DROWS), lambda s: (s, 0, 0)),
            pl.BlockSpec((NPAIR, D), lambda s: (0, 0)),
            pl.BlockSpec((NPAIR, D), lambda s: (1, 0)),
            pl.BlockSpec((NPAIR,), lambda s: (0,)),
        ],
        out_specs=pl.BlockSpec((1, 1), lambda s: (0, 0)),
        out_shape=jax.ShapeDtypeStruct((1, 1), jnp.float32),
        scratch_shapes=[pltpu.VMEM((NPAIR,), jnp.float32)],
    )(emb, p3, a_rows, p_rows, valid2)


def kernel(emb, labels, sbj):
    p = sbj * N_CLASSES + labels
    idx, valid2 = _mine(p.reshape(NBLK, ROWS))
    rows = _sc_gather(emb, idx)
    loss = _dist(emb, p.reshape(DNBLK, 1, DROWS), rows, rows, valid2)
    return loss.reshape(())


# speedup vs baseline: 285.1759x; 1.0696x over previous
import functools

import jax
import jax.numpy as jnp
from jax import lax
from jax.experimental import pallas as pl
from jax.experimental.pallas import tpu as pltpu
from jax.experimental.pallas import tpu_sc as plsc

B = 16384
D = 256
N_CLASSES = 6
N_SUBJECTS = 100
NPAIR = 640
MARGIN = 1.0
EPS = 1e-6
ROWS = 1024
NBLK = B // ROWS
DROWS = 2048
DNBLK = B // DROWS
I_SENT = 2**30
F_BIG = 1e30


def _mine_body(p_ref, idx_ref, valid2_ref):
    qi = lax.broadcasted_iota(jnp.int32, (ROWS, NPAIR), 1)
    jg = lax.broadcasted_iota(jnp.int32, (ROWS, NPAIR), 0)
    f = jnp.full((NPAIR,), I_SENT, jnp.int32)
    s = jnp.full((NPAIR,), I_SENT, jnp.int32)
    for r in range(NBLK):
        pr = p_ref[r, :][:, None]
        m = jnp.where(pr == qi, jg, I_SENT)
        c1 = jnp.min(m, axis=0)
        c2 = jnp.min(jnp.where(m == c1[None, :], I_SENT, m), axis=0)
        c1 = jnp.where(c1 < I_SENT, c1 + r * ROWS, I_SENT)
        c2 = jnp.where(c2 < I_SENT, c2 + r * ROWS, I_SENT)
        s = jnp.minimum(jnp.maximum(f, c1), jnp.minimum(s, c2))
        f = jnp.minimum(f, c1)
    idx_ref[0:NPAIR] = jnp.minimum(f, B - 1)
    idx_ref[NPAIR:2 * NPAIR] = jnp.minimum(s, B - 1)
    valid2_ref[...] = (s < I_SENT).astype(jnp.int32)


def _mine(p2d):
    return pl.pallas_call(
        _mine_body,
        out_shape=(
            jax.ShapeDtypeStruct((2 * NPAIR,), jnp.int32),
            jax.ShapeDtypeStruct((NPAIR,), jnp.int32),
        ),
    )(p2d)


_ROWS_PER_W = (2 * NPAIR) // 32


@functools.cache
def _make_sc_gather():
    mesh = plsc.VectorSubcoreMesh(core_axis_name="c", subcore_axis_name="s")

    @functools.partial(
        pl.kernel,
        mesh=mesh,
        out_type=jax.ShapeDtypeStruct((2 * NPAIR, D), jnp.float32),
        scratch_types=[
            pltpu.VMEM((_ROWS_PER_W,), jnp.int32),
            pltpu.VMEM((_ROWS_PER_W, D), jnp.float32),
            pltpu.SemaphoreType.DMA,
        ],
    )
    def _g(emb_hbm, idx_hbm, out_hbm, idx_v, rows_v, sem):
        wid = lax.axis_index("s") * 2 + lax.axis_index("c")
        base = wid * _ROWS_PER_W
        pltpu.sync_copy(idx_hbm.at[pl.ds(base, _ROWS_PER_W)], idx_v)
        pltpu.async_copy(emb_hbm.at[idx_v], rows_v, sem).wait()
        pltpu.sync_copy(rows_v, out_hbm.at[pl.ds(base, _ROWS_PER_W)])

    return _g


def _sc_gather(emb, idx):
    return _make_sc_gather()(emb, idx)


def _dist_body(emb_ref, p_ref, a_ref, pos_ref, valid2_ref, loss_ref, minacc,
               qi16, qs8):
    step = pl.program_id(0)

    @pl.when(step == 0)
    def _():
        minacc[...] = jnp.full((NPAIR,), F_BIG, jnp.float32)
        q = lax.broadcasted_iota(jnp.int32, (DROWS, NPAIR), 1)
        qi16[...] = q.astype(jnp.int16)
        qs8[...] = (q // N_CLASSES).astype(jnp.int8)

    E = emb_ref[...]
    Eb = E.astype(jnp.bfloat16)
    Ab = a_ref[...].astype(jnp.bfloat16)
    G = lax.dot_general(Eb, Ab, (((1,), (1,)), ((), ())),
                        preferred_element_type=jnp.float32)
    en = jnp.sum(E * E, axis=1).astype(jnp.bfloat16)
    val = en[:, None] - 2.0 * G.astype(jnp.bfloat16)
    pr = p_ref[0, 0, :]
    pr16 = pr.astype(jnp.int16)[:, None]
    ps8 = (pr // N_CLASSES).astype(jnp.int8)[:, None]
    mask = (ps8 == qs8[...]) & (pr16 != qi16[...])
    masked = jnp.where(mask, val, jnp.bfloat16(F_BIG))
    mstep = jnp.min(masked, axis=0).astype(jnp.float32)
    minacc[...] = jnp.minimum(minacc[...], mstep)

    @pl.when(step == DNBLK - 1)
    def _():
        A = a_ref[...]
        P = pos_ref[...]
        an2 = jnp.sum(A * A, axis=1)
        mn = minacc[...]
        d_an = jnp.sqrt(jnp.maximum(mn + an2, 0.0))
        dif = A - P + EPS
        d_ap = jnp.sqrt(jnp.sum(dif * dif, axis=1))
        valid = (valid2_ref[...] > 0) & (mn < 1e29)
        term = jnp.where(valid, jnp.maximum(d_ap - d_an + MARGIN, 0.0), 0.0)
        total = jnp.sum(term)
        count = jnp.sum(valid.astype(jnp.float32))
        loss = jnp.where(count > 0.0, total / jnp.maximum(count, 1.0),
                         jnp.float32(0.0))
        loss_ref[...] = jnp.broadcast_to(loss, (1, 1))


def _dist(emb, p3, a_rows, p_rows, valid2):
    return pl.pallas_call(
        _dist_body,
        grid=(DNBLK,),
        in_specs=[
            pl.BlockSpec((DROWS, D), lambda s: (s, 0)),
            pl.BlockSpec((1, 1, DROWS), lambda s: (s, 0, 0)),
            pl.BlockSpec((NPAIR, D), lambda s: (0, 0)),
            pl.BlockSpec((NPAIR, D), lambda s: (1, 0)),
            pl.BlockSpec((NPAIR,), lambda s: (0,)),
        ],
        out_specs=pl.BlockSpec((1, 1), lambda s: (0, 0)),
        out_shape=jax.ShapeDtypeStruct((1, 1), jnp.float32),
        scratch_shapes=[pltpu.VMEM((NPAIR,), jnp.float32),
                        pltpu.VMEM((DROWS, NPAIR), jnp.int16),
                        pltpu.VMEM((DROWS, NPAIR), jnp.int8)],
    )(emb, p3, a_rows, p_rows, valid2)


def kernel(emb, labels, sbj):
    p = sbj * N_CLASSES + labels
    idx, valid2 = _mine(p.reshape(NBLK, ROWS))
    rows = _sc_gather(emb, idx)
    loss = _dist(emb, p.reshape(DNBLK, 1, DROWS), rows, rows, valid2)
    return loss.reshape(())
